# baseline (speedup 1.0000x reference)
import jax
import jax.numpy as jnp
from jax import lax
from jax.experimental import pallas as pl
from jax.experimental.pallas import tpu as pltpu
from jax.experimental.pallas import tpu_sc as plsc

_TILE = 4096


def _tc_body(x_ref, wlab_ref, blab_ref, wenc_ref, benc_ref, wc_ref, bc_ref,
             routed_ref, idx_ref):
    i = pl.program_id(0)
    x = x_ref[...]
    lab = jnp.dot(x, wlab_ref[...], preferred_element_type=jnp.float32)
    lab = lab + blab_ref[...]
    y = jnp.argmax(lab, axis=-1).astype(jnp.int32)
    z = jnp.dot(x, wenc_ref[...], preferred_element_type=jnp.float32)
    z = z + benc_ref[...]
    logits = jnp.dot(z, wc_ref[...], preferred_element_type=jnp.float32)
    logits = logits + bc_ref[...]
    acc = jnp.zeros((logits.shape[0], 16), jnp.float32)
    for e in range(8):
        sl = logits[:, e * 16:(e + 1) * 16]
        acc = acc + jnp.where((y == e)[:, None], sl, 0.0)
    routed_ref[...] = acc
    n = i * _TILE + jax.lax.broadcasted_iota(jnp.int32, (logits.shape[0], 1), 0)
    idx_ref[...] = n * 8 + y[:, None]


def _tc_stage(x, W_lab, b_lab, W_enc, b_enc, Wc, bc):
    N, D = x.shape
    grid = (N // _TILE,)
    return pl.pallas_call(
        _tc_body,
        grid=grid,
        in_specs=[
            pl.BlockSpec((_TILE, D), lambda i: (i, 0)),
            pl.BlockSpec((D, 8), lambda i: (0, 0)),
            pl.BlockSpec((1, 8), lambda i: (0, 0)),
            pl.BlockSpec((D, 128), lambda i: (0, 0)),
            pl.BlockSpec((1, 128), lambda i: (0, 0)),
            pl.BlockSpec((128, 128), lambda i: (0, 0)),
            pl.BlockSpec((1, 128), lambda i: (0, 0)),
        ],
        out_specs=[
            pl.BlockSpec((_TILE, 16), lambda i: (i, 0)),
            pl.BlockSpec((_TILE, 1), lambda i: (i, 0)),
        ],
        out_shape=[
            jax.ShapeDtypeStruct((N, 16), jnp.float32),
            jax.ShapeDtypeStruct((N, 1), jnp.int32),
        ],
    )(x, W_lab, b_lab[None, :], W_enc, b_enc[None, :], Wc, bc)


def _sc_stage(routed, idx2, zeros_hbm, N):
    info = plsc.get_sparse_core_info()
    NC, NS = info.num_cores, info.num_subcores
    NW = NC * NS
    chunk = N // NW
    rows_w = chunk * 8
    zrows = zeros_hbm.shape[0]
    idx_rows = chunk // 128
    mesh = plsc.VectorSubcoreMesh(core_axis_name="c", subcore_axis_name="s")

    def body(routed_hbm, idx_hbm, z_hbm, out_hbm, idx_v, routed_v, zeros_v,
             sem):
        wid = lax.axis_index("s") * NC + lax.axis_index("c")
        tok0 = wid * chunk
        row0 = wid * rows_w
        pltpu.sync_copy(idx_hbm.at[pl.ds(wid * idx_rows, idx_rows)], idx_v)
        pltpu.sync_copy(routed_hbm.at[pl.ds(tok0, chunk)], routed_v)
        pltpu.sync_copy(z_hbm, zeros_v)
        for k in range(rows_w // zrows):
            pltpu.sync_copy(zeros_v, out_hbm.at[pl.ds(row0 + k * zrows, zrows)])
        for j in range(idx_rows):
            pltpu.async_copy(routed_v.at[pl.ds(j * 128, 128)],
                             out_hbm.at[idx_v.at[j]], sem).wait()

    return pl.kernel(
        body,
        out_type=jax.ShapeDtypeStruct((N * 8, 16), jnp.float32),
        mesh=mesh,
        scratch_types=[
            pltpu.VMEM((idx_rows, 128), jnp.int32),
            pltpu.VMEM((chunk, 16), jnp.float32),
            pltpu.VMEM((zrows, 16), jnp.float32),
            pltpu.SemaphoreType.DMA,
        ],
        compiler_params=pltpu.CompilerParams(use_tc_tiling_on_sc=False),
    )(routed, idx2, zeros_hbm)


def kernel(x, W_lab, b_lab, W_enc, b_enc, W_clf, b_clf):
    N, D = x.shape
    E, H, S = W_clf.shape
    Wc = jnp.transpose(W_clf, (1, 0, 2)).reshape(H, E * S)
    bc = b_clf.reshape(1, E * S)
    routed, idx = _tc_stage(x, W_lab, b_lab, W_enc, b_enc, Wc, bc)
    idx2 = idx.reshape(N // 128, 128)
    zeros_hbm = jnp.zeros((1024, S), jnp.float32)
    out2 = _sc_stage(routed, idx2, zeros_hbm, N)
    return out2.reshape(N, E * S)


# speedup vs baseline: 1.1884x; 1.1884x over previous
import jax
import jax.numpy as jnp
from jax import lax
from jax.experimental import pallas as pl
from jax.experimental.pallas import tpu as pltpu
from jax.experimental.pallas import tpu_sc as plsc

_TILE = 4096


def _tc_body(x_ref, wlab_ref, blab_ref, wenc_ref, benc_ref, wc_ref, bc_ref,
             g_ref, routed_ref, idx_ref):
    i = pl.program_id(0)
    x = x_ref[...]
    lab = jnp.dot(x, wlab_ref[...], preferred_element_type=jnp.float32)
    lab = lab + blab_ref[...]
    y = jnp.argmax(lab, axis=-1).astype(jnp.int32)
    z = jnp.dot(x, wenc_ref[...], preferred_element_type=jnp.float32)
    z = z + benc_ref[...]
    logits = jnp.dot(z, wc_ref[...], preferred_element_type=jnp.float32)
    logits = logits + bc_ref[...]
    expert_of_col = jax.lax.broadcasted_iota(jnp.int32, logits.shape, 1) // 16
    masked = jnp.where(expert_of_col == y[:, None], logits, 0.0)
    routed_ref[...] = jnp.dot(masked, g_ref[...],
                              preferred_element_type=jnp.float32)
    n = i * _TILE + jax.lax.broadcasted_iota(jnp.int32, (logits.shape[0], 1), 0)
    idx_ref[...] = n * 8 + y[:, None]


def _tc_stage(x, W_lab, b_lab, W_enc, b_enc, Wc, bc, G):
    N, D = x.shape
    grid = (N // _TILE,)
    return pl.pallas_call(
        _tc_body,
        grid=grid,
        in_specs=[
            pl.BlockSpec((_TILE, D), lambda i: (i, 0)),
            pl.BlockSpec((D, 8), lambda i: (0, 0)),
            pl.BlockSpec((1, 8), lambda i: (0, 0)),
            pl.BlockSpec((D, 128), lambda i: (0, 0)),
            pl.BlockSpec((1, 128), lambda i: (0, 0)),
            pl.BlockSpec((128, 128), lambda i: (0, 0)),
            pl.BlockSpec((1, 128), lambda i: (0, 0)),
            pl.BlockSpec((128, 16), lambda i: (0, 0)),
        ],
        out_specs=[
            pl.BlockSpec((_TILE, 16), lambda i: (i, 0)),
            pl.BlockSpec((_TILE, 1), lambda i: (i, 0)),
        ],
        out_shape=[
            jax.ShapeDtypeStruct((N, 16), jnp.float32),
            jax.ShapeDtypeStruct((N, 1), jnp.int32),
        ],
    )(x, W_lab, b_lab[None, :], W_enc, b_enc[None, :], Wc, bc, G)


def _sc_stage(routed, idx2, zeros_hbm, N):
    info = plsc.get_sparse_core_info()
    NC, NS = info.num_cores, info.num_subcores
    NW = NC * NS
    chunk = N // NW
    rows_w = chunk * 8
    zrows = zeros_hbm.shape[0]
    idx_rows = chunk // 128
    mesh = plsc.VectorSubcoreMesh(core_axis_name="c", subcore_axis_name="s")

    def body(routed_hbm, idx_hbm, z_hbm, out_hbm, idx_v, routed_v, zeros_v,
             sem):
        wid = lax.axis_index("s") * NC + lax.axis_index("c")
        tok0 = wid * chunk
        row0 = wid * rows_w
        pltpu.sync_copy(idx_hbm.at[pl.ds(wid * idx_rows, idx_rows)], idx_v)
        pltpu.sync_copy(routed_hbm.at[pl.ds(tok0, chunk)], routed_v)
        pltpu.sync_copy(z_hbm, zeros_v)
        zfills = [
            pltpu.async_copy(zeros_v,
                             out_hbm.at[pl.ds(row0 + k * zrows, zrows)], sem)
            for k in range(rows_w // zrows)
        ]
        for d in zfills:
            d.wait()
        scats = [
            pltpu.async_copy(routed_v.at[pl.ds(j * 128, 128)],
                             out_hbm.at[idx_v.at[j]], sem)
            for j in range(idx_rows)
        ]
        for d in scats:
            d.wait()

    return pl.kernel(
        body,
        out_type=jax.ShapeDtypeStruct((N * 8, 16), jnp.float32),
        mesh=mesh,
        scratch_types=[
            pltpu.VMEM((idx_rows, 128), jnp.int32),
            pltpu.VMEM((chunk, 16), jnp.float32),
            pltpu.VMEM((zeros_hbm.shape[0], 16), jnp.float32),
            pltpu.SemaphoreType.DMA,
        ],
        compiler_params=pltpu.CompilerParams(use_tc_tiling_on_sc=False),
    )(routed, idx2, zeros_hbm)


def kernel(x, W_lab, b_lab, W_enc, b_enc, W_clf, b_clf):
    N, D = x.shape
    E, H, S = W_clf.shape
    Wc = jnp.transpose(W_clf, (1, 0, 2)).reshape(H, E * S)
    bc = b_clf.reshape(1, E * S)
    cols = jnp.arange(E * S, dtype=jnp.int32)
    G = (cols[:, None] % S == jnp.arange(S, dtype=jnp.int32)[None, :])
    G = G.astype(jnp.float32)
    routed, idx = _tc_stage(x, W_lab, b_lab, W_enc, b_enc, Wc, bc, G)
    idx2 = idx.reshape(N // 128, 128)
    zeros_hbm = jnp.zeros((2048, S), jnp.float32)
    out2 = _sc_stage(routed, idx2, zeros_hbm, N)
    return out2.reshape(N, E * S)


# speedup vs baseline: 2.6577x; 2.2364x over previous
import jax
import jax.numpy as jnp
from jax.experimental import pallas as pl

_TILE = 4096


def _fused_body(x_ref, wlab_ref, blab_ref, wenc_ref, benc_ref, wc_ref, bc_ref,
                out_ref):
    x = x_ref[...]
    lab = jnp.dot(x, wlab_ref[...], preferred_element_type=jnp.float32)
    lab = lab + blab_ref[...]
    y = jnp.argmax(lab, axis=-1)
    z = jnp.dot(x, wenc_ref[...], preferred_element_type=jnp.float32)
    z = z + benc_ref[...]
    logits = jnp.dot(z, wc_ref[...], preferred_element_type=jnp.float32)
    logits = logits + bc_ref[...]
    expert_of_col = jax.lax.broadcasted_iota(jnp.int32, logits.shape, 1) // 16
    out_ref[...] = jnp.where(expert_of_col == y[:, None], logits, 0.0)


def kernel(x, W_lab, b_lab, W_enc, b_enc, W_clf, b_clf):
    N, D = x.shape
    E, H, S = W_clf.shape
    Wc = jnp.transpose(W_clf, (1, 0, 2)).reshape(H, E * S)
    bc = b_clf.reshape(1, E * S)
    grid = (N // _TILE,)
    return pl.pallas_call(
        _fused_body,
        grid=grid,
        in_specs=[
            pl.BlockSpec((_TILE, D), lambda i: (i, 0)),
            pl.BlockSpec((D, E), lambda i: (0, 0)),
            pl.BlockSpec((1, E), lambda i: (0, 0)),
            pl.BlockSpec((D, H), lambda i: (0, 0)),
            pl.BlockSpec((1, H), lambda i: (0, 0)),
            pl.BlockSpec((H, E * S), lambda i: (0, 0)),
            pl.BlockSpec((1, E * S), lambda i: (0, 0)),
        ],
        out_specs=pl.BlockSpec((_TILE, E * S), lambda i: (i, 0)),
        out_shape=jax.ShapeDtypeStruct((N, E * S), x.dtype),
    )(x, W_lab, b_lab[None, :], W_enc, b_enc[None, :], Wc, bc)
